# baseline (speedup 1.0000x reference)
import functools

import jax
import jax.numpy as jnp
from jax import lax
from jax.experimental import pallas as pl
from jax.experimental.pallas import tpu as pltpu
from jax.experimental.pallas import tpu_sc as plsc

B = 256
N0 = 1024
N1 = 512
NK = 1025
NH = 12

NUM_CORES = 2
NUM_SUBCORES = 16
NUM_WORKERS = NUM_CORES * NUM_SUBCORES
LANES = 16

ROWS_PER_WORKER = B // NUM_WORKERS
CHUNKS = N1 // LANES


def _tc_renorm_body(w_ref, p0_ref, renorm_ref, ssq_ref, acc):
    j = pl.program_id(0)
    s = jnp.max(jnp.sum(w_ref[...], axis=0), axis=0)

    @pl.when(j == 0)
    def _():
        acc[...] = s

    @pl.when(j != 0)
    def _():
        acc[...] = jnp.maximum(acc[...], s)

    @pl.when(j == NH // 2 - 1)
    def _():
        cls = acc[pl.ds(1, N0), :]
        denom = jnp.sum(cls, axis=0, keepdims=True)
        renorm = jnp.transpose(cls / denom)
        renorm_ref[...] = renorm
        d = p0_ref[...] - renorm
        ssq_ref[0, 0] = jnp.sum(d * d)


def _tc_renorm_loss0(w4, p0):
    return pl.pallas_call(
        _tc_renorm_body,
        grid=(NH // 2,),
        in_specs=[
            pl.BlockSpec((NH, 2, NK, B), lambda j: (0, j, 0, 0)),
            pl.BlockSpec((B, N0), lambda j: (0, 0)),
        ],
        out_specs=[
            pl.BlockSpec((B, N0), lambda j: (0, 0)),
            pl.BlockSpec(block_shape=(1, 1), index_map=lambda j: (0, 0),
                         memory_space=pltpu.SMEM),
        ],
        out_shape=[
            jax.ShapeDtypeStruct((B, N0), jnp.float32),
            jax.ShapeDtypeStruct((1, 1), jnp.float32),
        ],
        scratch_shapes=[pltpu.VMEM((NK, B), jnp.float32)],
    )(w4, p0)


def _sc_gather_body(renorm_hbm, idx_hbm, p1_hbm, out_hbm,
                    row_v, idx_v, p1_v, g_v, acc_v):
    wid = lax.axis_index("s") * NUM_CORES + lax.axis_index("c")
    base = wid * ROWS_PER_WORKER
    pltpu.sync_copy(renorm_hbm.at[pl.ds(base * N0, ROWS_PER_WORKER * N0)], row_v)
    pltpu.sync_copy(idx_hbm.at[pl.ds(base * N1, ROWS_PER_WORKER * N1)], idx_v)
    pltpu.sync_copy(p1_hbm.at[pl.ds(base * N1, ROWS_PER_WORKER * N1)], p1_v)
    acc = jnp.zeros((LANES,), jnp.float32)
    for r in range(ROWS_PER_WORKER):
        s = jnp.zeros((LANES,), jnp.float32)
        for j in range(CHUNKS):
            iv = idx_v[pl.ds(r * N1 + j * LANES, LANES)] + jnp.int32(r * N0)
            g = plsc.load_gather(row_v, [iv])
            g_v[pl.ds(j * LANES, LANES)] = g
            s = s + g
        total_v = lax.broadcast(jnp.sum(s), (LANES,))
        inv_v = jnp.ones((LANES,), jnp.float32) / total_v
        for j in range(CHUNKS):
            d = (p1_v[pl.ds(r * N1 + j * LANES, LANES)]
                 - g_v[pl.ds(j * LANES, LANES)] * inv_v)
            acc = acc + d * d
    acc_v[...] = acc
    pltpu.sync_copy(acc_v, out_hbm.at[pl.ds(wid * LANES, LANES)])


@functools.cache
def _sc_gather_loss1():
    return pl.kernel(
        _sc_gather_body,
        mesh=plsc.VectorSubcoreMesh(core_axis_name="c", subcore_axis_name="s"),
        out_type=jax.ShapeDtypeStruct((NUM_WORKERS * LANES,), jnp.float32),
        scratch_types=[
            pltpu.VMEM((ROWS_PER_WORKER * N0,), jnp.float32),
            pltpu.VMEM((ROWS_PER_WORKER * N1,), jnp.int32),
            pltpu.VMEM((ROWS_PER_WORKER * N1,), jnp.float32),
            pltpu.VMEM((N1,), jnp.float32),
            pltpu.VMEM((LANES,), jnp.float32),
        ],
        compiler_params=pltpu.CompilerParams(needs_layout_passes=False),
    )


def kernel(pred_logits_0, pred_logits_1, cls_attn_weights,
           kept_token_idx_0, kept_token_idx_1):
    w4 = jnp.transpose(cls_attn_weights, (1, 2, 3, 0))
    renorm, ssq0 = _tc_renorm_loss0(w4, pred_logits_0)
    partials = _sc_gather_loss1()(renorm.reshape(-1),
                                  kept_token_idx_0.reshape(-1),
                                  pred_logits_1.reshape(-1))
    loss0 = 100.0 * ssq0[0, 0] / (B * N0)
    loss1 = 100.0 * jnp.sum(partials) / (B * N1)
    return loss0 + loss1


# speedup vs baseline: 1.0190x; 1.0190x over previous
import functools

import jax
import jax.numpy as jnp
from jax import lax
from jax.experimental import pallas as pl
from jax.experimental.pallas import tpu as pltpu
from jax.experimental.pallas import tpu_sc as plsc

B = 256
N0 = 1024
N1 = 512
NK = 1025
NH = 12

NUM_CORES = 2
NUM_SUBCORES = 16
NUM_WORKERS = NUM_CORES * NUM_SUBCORES
LANES = 16

ROWS_PER_WORKER = B // NUM_WORKERS
CHUNKS = N1 // LANES


def _tc_renorm_body(w_ref, p0_ref, renorm_ref, ssq_ref, acc):
    j = pl.program_id(0)
    s = jnp.max(jnp.sum(w_ref[...], axis=0), axis=0)

    @pl.when(j == 0)
    def _():
        acc[...] = s

    @pl.when(j != 0)
    def _():
        acc[...] = jnp.maximum(acc[...], s)

    @pl.when(j == NH // 2 - 1)
    def _():
        cls = acc[pl.ds(1, N0), :]
        denom = jnp.sum(cls, axis=0, keepdims=True)
        renorm = jnp.transpose(cls / denom)
        renorm_ref[...] = renorm
        d = p0_ref[...] - renorm
        ssq_ref[0, 0] = jnp.sum(d * d)


def _tc_renorm_loss0(w4, p0):
    return pl.pallas_call(
        _tc_renorm_body,
        grid=(NH // 2,),
        in_specs=[
            pl.BlockSpec((NH, 2, NK, B), lambda j: (0, j, 0, 0)),
            pl.BlockSpec((B, N0), lambda j: (0, 0)),
        ],
        out_specs=[
            pl.BlockSpec((B, N0), lambda j: (0, 0)),
            pl.BlockSpec(block_shape=(1, 1), index_map=lambda j: (0, 0),
                         memory_space=pltpu.SMEM),
        ],
        out_shape=[
            jax.ShapeDtypeStruct((B, N0), jnp.float32),
            jax.ShapeDtypeStruct((1, 1), jnp.float32),
        ],
        scratch_shapes=[pltpu.VMEM((NK, B), jnp.float32)],
    )(w4, p0)


def _sc_gather_body(renorm_hbm, idx_hbm, p1_hbm, out_hbm,
                    row_v, idx_v, p1_v, g_v, acc_v, sem):
    wid = lax.axis_index("s") * NUM_CORES + lax.axis_index("c")
    base = wid * ROWS_PER_WORKER
    h1 = pltpu.async_copy(
        renorm_hbm.at[pl.ds(base * N0, ROWS_PER_WORKER * N0)], row_v, sem)
    h2 = pltpu.async_copy(
        idx_hbm.at[pl.ds(base * N1, ROWS_PER_WORKER * N1)], idx_v, sem)
    h3 = pltpu.async_copy(
        p1_hbm.at[pl.ds(base * N1, ROWS_PER_WORKER * N1)], p1_v, sem)
    h1.wait()
    h2.wait()
    h3.wait()
    acc = jnp.zeros((LANES,), jnp.float32)
    for r in range(ROWS_PER_WORKER):
        s = jnp.zeros((LANES,), jnp.float32)
        for j in range(CHUNKS):
            iv = idx_v[pl.ds(r * N1 + j * LANES, LANES)] + jnp.int32(r * N0)
            g = plsc.load_gather(row_v, [iv])
            g_v[pl.ds(j * LANES, LANES)] = g
            s = s + g
        total_v = lax.broadcast(jnp.sum(s), (LANES,))
        inv_v = jnp.ones((LANES,), jnp.float32) / total_v
        for j in range(CHUNKS):
            d = (p1_v[pl.ds(r * N1 + j * LANES, LANES)]
                 - g_v[pl.ds(j * LANES, LANES)] * inv_v)
            acc = acc + d * d
    acc_v[...] = acc
    pltpu.sync_copy(acc_v, out_hbm.at[pl.ds(wid * LANES, LANES)])


@functools.cache
def _sc_gather_loss1():
    return pl.kernel(
        _sc_gather_body,
        mesh=plsc.VectorSubcoreMesh(core_axis_name="c", subcore_axis_name="s"),
        out_type=jax.ShapeDtypeStruct((NUM_WORKERS * LANES,), jnp.float32),
        scratch_types=[
            pltpu.VMEM((ROWS_PER_WORKER * N0,), jnp.float32),
            pltpu.VMEM((ROWS_PER_WORKER * N1,), jnp.int32),
            pltpu.VMEM((ROWS_PER_WORKER * N1,), jnp.float32),
            pltpu.VMEM((N1,), jnp.float32),
            pltpu.VMEM((LANES,), jnp.float32),
            pltpu.SemaphoreType.DMA,
        ],
        compiler_params=pltpu.CompilerParams(needs_layout_passes=False),
    )


def kernel(pred_logits_0, pred_logits_1, cls_attn_weights,
           kept_token_idx_0, kept_token_idx_1):
    w4 = jnp.transpose(cls_attn_weights, (1, 2, 3, 0))
    renorm, ssq0 = _tc_renorm_loss0(w4, pred_logits_0)
    partials = _sc_gather_loss1()(renorm.reshape(-1),
                                  kept_token_idx_0.reshape(-1),
                                  pred_logits_1.reshape(-1))
    loss0 = 100.0 * ssq0[0, 0] / (B * N0)
    loss1 = 100.0 * jnp.sum(partials) / (B * N1)
    return loss0 + loss1
